# baseline (speedup 1.0000x reference)
import functools
import jax
import jax.numpy as jnp
from jax import lax
from jax.experimental import pallas as pl
from jax.experimental.pallas import tpu as pltpu
from jax.experimental.pallas import tpu_sc as plsc

N_DIM = 128
HIDDEN = 256
K = 8
EPS = 1e-11


def _sc_select_kernel(sc2_hbm, wsel_hbm, rows_v, out_v):
    f32 = jnp.float32
    N2 = out_v.shape[0]
    D = rows_v.shape[0] // N2
    b = lax.axis_index("s") * 2 + lax.axis_index("c")

    pltpu.sync_copy(sc2_hbm.at[b], rows_v)

    iota16 = lax.iota(jnp.int32, 16)
    nchunks = N2 // 16

    scales = []
    for c in range(nchunks):
        sq = jnp.zeros((16,), f32)
        for i in range(16):
            n = c * 16 + i
            acc = jnp.zeros((16,), f32)
            for j in range(D // 16):
                x = rows_v[n * D + j * 16:n * D + (j + 1) * 16]
                acc = acc + x * x
            sq = jnp.where(iota16 == i, jnp.sum(acc), sq)
        scales.append(sq / (1.0 + sq))

    msk = list(scales)
    keep = [jnp.zeros((16,), f32) for _ in range(nchunks)]
    for _ in range(K):
        m = jnp.max(msk[0])
        for c in range(1, nchunks):
            m = jnp.maximum(m, jnp.max(msk[c]))
        tot = jnp.float32(0.0)
        for c in range(nchunks):
            eq = (msk[c] == m).astype(f32)
            cs = plsc.cumsum(eq)
            first = (eq > 0.0) & (cs + tot == 1.0)
            keep[c] = jnp.where(first, 1.0, keep[c])
            msk[c] = jnp.where(first, -1.0, msk[c])
            tot = tot + jnp.sum(eq)
    for c in range(nchunks):
        out_v[c * 16:(c + 1) * 16] = scales[c] * keep[c]

    pltpu.sync_copy(out_v, wsel_hbm.at[b])


def _select_kernel(sc2_ref, cls_ref, y_ref, wsel_ref, w_ref, b_ref,
                   v_ref, s_ref):
    f32 = jnp.float32
    B, N2, D = sc2_ref.shape
    NC = cls_ref.shape[1]

    sc2 = sc2_ref[...]
    ones_d = jnp.ones((D, 1), f32)
    sq = lax.dot_general(jnp.reshape(sc2 * sc2, (B * N2, D)), ones_d,
                         (((1,), (0,)), ((), ())),
                         preferred_element_type=f32)
    sq3 = jnp.reshape(sq, (B, N2, 1))
    sc2n = sc2 / jnp.sqrt(sq3 + EPS)

    wsel = wsel_ref[...]

    iota_c = lax.broadcasted_iota(jnp.int32, (B, NC, 1), 1)
    y3 = jnp.reshape(y_ref[...], (B, 1, 1))
    y1h = (iota_c == y3).astype(f32)
    cc_raw = jnp.sum(y1h * cls_ref[...], axis=1)
    sqc = lax.dot_general(cc_raw * cc_raw, ones_d, (((1,), (0,)), ((), ())),
                          preferred_element_type=f32)
    ccn = cc_raw / jnp.sqrt(sqc + EPS)
    scale1 = sqc / (1.0 + sqc)

    fw = w_ref[...]
    bias = b_ref[...]
    sproc = jnp.maximum(
        lax.dot_general(jnp.reshape(sc2n, (B * N2, D)), fw,
                        (((1,), (1,)), ((), ())),
                        preferred_element_type=f32) + bias, 0.0)
    s1 = jnp.maximum(
        lax.dot_general(ccn, fw, (((1,), (1,)), ((), ())),
                        preferred_element_type=f32) + bias, 0.0)

    ones_h = jnp.ones((HIDDEN, 1), f32)
    rowsq = lax.dot_general(sproc * sproc, ones_h, (((1,), (0,)), ((), ())),
                            preferred_element_type=f32)
    s1sq = lax.dot_general(s1 * s1, ones_h, (((1,), (0,)), ((), ())),
                           preferred_element_type=f32)

    sproc3 = jnp.reshape(sproc, (B, N2, HIDDEN))
    rowsq3 = jnp.reshape(rowsq, (B, N2, 1))

    v = jnp.sum(wsel * sproc3, axis=1) + scale1 * s1
    Wt = jnp.sum(wsel, axis=1) + scale1
    c = jnp.sum(wsel * rowsq3, axis=1) + scale1 * s1sq

    v_ref[...] = jnp.reshape(v, (B, 1, HIDDEN))
    pad = jnp.zeros((B, 126), f32)
    s_ref[...] = jnp.reshape(
        jnp.concatenate([Wt, c, pad], axis=1), (B, 1, 128))


def _dense_kernel(fcap_ref, v_ref, s_ref, mask_ref, out_ref):
    f32 = jnp.float32
    BB = fcap_ref.shape[0]
    for i in range(BB):
        F = jnp.maximum(fcap_ref[i], 0.0)
        v = v_ref[i]
        W = s_ref[i][0:1, 0:1]
        c = s_ref[i][0:1, 1:2]

        G = lax.dot_general(F, F, (((1,), (1,)), ((), ())),
                            preferred_element_type=f32)
        u_col = lax.dot_general(F, v, (((1,), (1,)), ((), ())),
                                preferred_element_type=f32)
        u_row = lax.dot_general(v, F, (((1,), (1,)), ((), ())),
                                preferred_element_type=f32)

        adj = W * G + u_col + u_row + c
        out_ref[i] = adj * mask_ref[i]


def kernel(first_capsule, second_capsule, class_capsule, fc1_w, fc1_b, y,
           first_capsule_mask):
    B, M, H = first_capsule.shape
    N2 = second_capsule.shape[1]
    NC = class_capsule.shape[1]
    f32 = jnp.float32

    y2 = y.astype(jnp.int32).reshape(B, 1)
    mask3 = first_capsule_mask.astype(f32).reshape(B, M, 1)
    fc1_b2 = fc1_b.reshape(1, H)

    sc_select = functools.partial(
        pl.kernel,
        mesh=plsc.VectorSubcoreMesh(core_axis_name="c", subcore_axis_name="s"),
        out_type=jax.ShapeDtypeStruct((B, N2), f32),
        scratch_types=[
            pltpu.VMEM((N2 * N_DIM,), f32),
            pltpu.VMEM((N2,), f32),
        ],
        compiler_params=pltpu.CompilerParams(needs_layout_passes=False),
    )(_sc_select_kernel)
    wsel = sc_select(second_capsule.reshape(B, N2 * N_DIM))
    wsel3 = wsel.reshape(B, N2, 1)

    v_all, s_all = pl.pallas_call(
        _select_kernel,
        in_specs=[
            pl.BlockSpec(second_capsule.shape, lambda: (0, 0, 0)),
            pl.BlockSpec(class_capsule.shape, lambda: (0, 0, 0)),
            pl.BlockSpec((B, 1), lambda: (0, 0)),
            pl.BlockSpec((B, N2, 1), lambda: (0, 0, 0)),
            pl.BlockSpec(fc1_w.shape, lambda: (0, 0)),
            pl.BlockSpec(fc1_b2.shape, lambda: (0, 0)),
        ],
        out_specs=[
            pl.BlockSpec((B, 1, H), lambda: (0, 0, 0)),
            pl.BlockSpec((B, 1, 128), lambda: (0, 0, 0)),
        ],
        out_shape=[
            jax.ShapeDtypeStruct((B, 1, H), f32),
            jax.ShapeDtypeStruct((B, 1, 128), f32),
        ],
    )(second_capsule, class_capsule, y2, wsel3, fc1_w, fc1_b2)

    BB = 8
    out = pl.pallas_call(
        _dense_kernel,
        grid=(B // BB,),
        in_specs=[
            pl.BlockSpec((BB, M, H), lambda b: (b, 0, 0)),
            pl.BlockSpec((BB, 1, H), lambda b: (b, 0, 0)),
            pl.BlockSpec((BB, 1, 128), lambda b: (b, 0, 0)),
            pl.BlockSpec((BB, M, 1), lambda b: (b, 0, 0)),
        ],
        out_specs=pl.BlockSpec((BB, M, M), lambda b: (b, 0, 0)),
        out_shape=jax.ShapeDtypeStruct((B, M, M), f32),
        compiler_params=pltpu.CompilerParams(
            dimension_semantics=("arbitrary",)),
    )(first_capsule, v_all, s_all, mask3)
    return out


# speedup vs baseline: 1.0130x; 1.0130x over previous
import functools
import jax
import jax.numpy as jnp
from jax import lax
from jax.experimental import pallas as pl
from jax.experimental.pallas import tpu as pltpu
from jax.experimental.pallas import tpu_sc as plsc

N_DIM = 128
HIDDEN = 256
K = 8
EPS = 1e-11


def _sc_select_kernel(sc2_hbm, wsel_hbm, rows_v, out_v):
    nb = sc2_hbm.shape[0] // 16
    s_id = lax.axis_index("s")

    for bi in range(nb):
        b = s_id * nb + bi
        _select_one(sc2_hbm, wsel_hbm, rows_v, out_v, b)


def _select_one(sc2_hbm, wsel_hbm, rows_v, out_v, b):
    f32 = jnp.float32
    N2 = out_v.shape[0]
    D = rows_v.shape[0] // N2

    pltpu.sync_copy(sc2_hbm.at[b], rows_v)

    iota16 = lax.iota(jnp.int32, 16)
    nchunks = N2 // 16

    scales = []
    for c in range(nchunks):
        sq = jnp.zeros((16,), f32)
        for i in range(16):
            n = c * 16 + i
            acc = jnp.zeros((16,), f32)
            for j in range(D // 16):
                x = rows_v[n * D + j * 16:n * D + (j + 1) * 16]
                acc = acc + x * x
            sq = jnp.where(iota16 == i, jnp.sum(acc), sq)
        scales.append(sq / (1.0 + sq))

    msk = list(scales)
    keep = [jnp.zeros((16,), f32) for _ in range(nchunks)]
    for _ in range(K):
        m = jnp.max(msk[0])
        for c in range(1, nchunks):
            m = jnp.maximum(m, jnp.max(msk[c]))
        tot = jnp.float32(0.0)
        for c in range(nchunks):
            eq = (msk[c] == m).astype(f32)
            cs = plsc.cumsum(eq)
            first = (eq > 0.0) & (cs + tot == 1.0)
            keep[c] = jnp.where(first, 1.0, keep[c])
            msk[c] = jnp.where(first, -1.0, msk[c])
            tot = tot + jnp.sum(eq)
    for c in range(nchunks):
        out_v[c * 16:(c + 1) * 16] = scales[c] * keep[c]

    pltpu.sync_copy(out_v, wsel_hbm.at[b])


def _select_kernel(sc2_ref, cls_ref, y_ref, wsel_ref, w_ref, b_ref,
                   v_ref, s_ref):
    f32 = jnp.float32
    B, N2, D = sc2_ref.shape
    NC = cls_ref.shape[1]

    sc2 = sc2_ref[...]
    ones_d = jnp.ones((D, 1), f32)
    sq = lax.dot_general(jnp.reshape(sc2 * sc2, (B * N2, D)), ones_d,
                         (((1,), (0,)), ((), ())),
                         preferred_element_type=f32)
    sq3 = jnp.reshape(sq, (B, N2, 1))
    sc2n = sc2 / jnp.sqrt(sq3 + EPS)

    wsel = wsel_ref[...]

    iota_c = lax.broadcasted_iota(jnp.int32, (B, NC, 1), 1)
    y3 = jnp.reshape(y_ref[...], (B, 1, 1))
    y1h = (iota_c == y3).astype(f32)
    cc_raw = jnp.sum(y1h * cls_ref[...], axis=1)
    sqc = lax.dot_general(cc_raw * cc_raw, ones_d, (((1,), (0,)), ((), ())),
                          preferred_element_type=f32)
    ccn = cc_raw / jnp.sqrt(sqc + EPS)
    scale1 = sqc / (1.0 + sqc)

    fw = w_ref[...]
    bias = b_ref[...]
    sproc = jnp.maximum(
        lax.dot_general(jnp.reshape(sc2n, (B * N2, D)), fw,
                        (((1,), (1,)), ((), ())),
                        preferred_element_type=f32) + bias, 0.0)
    s1 = jnp.maximum(
        lax.dot_general(ccn, fw, (((1,), (1,)), ((), ())),
                        preferred_element_type=f32) + bias, 0.0)

    ones_h = jnp.ones((HIDDEN, 1), f32)
    rowsq = lax.dot_general(sproc * sproc, ones_h, (((1,), (0,)), ((), ())),
                            preferred_element_type=f32)
    s1sq = lax.dot_general(s1 * s1, ones_h, (((1,), (0,)), ((), ())),
                           preferred_element_type=f32)

    sproc3 = jnp.reshape(sproc, (B, N2, HIDDEN))
    rowsq3 = jnp.reshape(rowsq, (B, N2, 1))

    v = jnp.sum(wsel * sproc3, axis=1) + scale1 * s1
    Wt = jnp.sum(wsel, axis=1) + scale1
    c = jnp.sum(wsel * rowsq3, axis=1) + scale1 * s1sq

    v_ref[...] = jnp.reshape(v, (B, 1, HIDDEN))
    pad = jnp.zeros((B, 126), f32)
    s_ref[...] = jnp.reshape(
        jnp.concatenate([Wt, c, pad], axis=1), (B, 1, 128))


def _dense_kernel(fcap_ref, v_ref, s_ref, mask_ref, out_ref):
    f32 = jnp.float32
    BB = fcap_ref.shape[0]
    for i in range(BB):
        F = jnp.maximum(fcap_ref[i], 0.0)
        v = v_ref[i]
        W = s_ref[i][0:1, 0:1]
        c = s_ref[i][0:1, 1:2]

        G = lax.dot_general(F, F, (((1,), (1,)), ((), ())),
                            preferred_element_type=f32)
        u_col = lax.dot_general(F, v, (((1,), (1,)), ((), ())),
                                preferred_element_type=f32)
        u_row = lax.dot_general(v, F, (((1,), (1,)), ((), ())),
                                preferred_element_type=f32)

        adj = W * G + u_col + u_row + c
        out_ref[i] = adj * mask_ref[i]


def kernel(first_capsule, second_capsule, class_capsule, fc1_w, fc1_b, y,
           first_capsule_mask):
    B, M, H = first_capsule.shape
    N2 = second_capsule.shape[1]
    NC = class_capsule.shape[1]
    f32 = jnp.float32

    y2 = y.astype(jnp.int32).reshape(B, 1)
    mask3 = first_capsule_mask.astype(f32).reshape(B, M, 1)
    fc1_b2 = fc1_b.reshape(1, H)

    sc_select = functools.partial(
        pl.kernel,
        mesh=plsc.VectorSubcoreMesh(core_axis_name="c", subcore_axis_name="s",
                                    num_cores=1),
        out_type=jax.ShapeDtypeStruct((B, N2), f32),
        scratch_types=[
            pltpu.VMEM((N2 * N_DIM,), f32),
            pltpu.VMEM((N2,), f32),
        ],
        compiler_params=pltpu.CompilerParams(needs_layout_passes=False),
    )(_sc_select_kernel)
    wsel = sc_select(second_capsule.reshape(B, N2 * N_DIM))
    wsel3 = wsel.reshape(B, N2, 1)

    v_all, s_all = pl.pallas_call(
        _select_kernel,
        in_specs=[
            pl.BlockSpec(second_capsule.shape, lambda: (0, 0, 0)),
            pl.BlockSpec(class_capsule.shape, lambda: (0, 0, 0)),
            pl.BlockSpec((B, 1), lambda: (0, 0)),
            pl.BlockSpec((B, N2, 1), lambda: (0, 0, 0)),
            pl.BlockSpec(fc1_w.shape, lambda: (0, 0)),
            pl.BlockSpec(fc1_b2.shape, lambda: (0, 0)),
        ],
        out_specs=[
            pl.BlockSpec((B, 1, H), lambda: (0, 0, 0)),
            pl.BlockSpec((B, 1, 128), lambda: (0, 0, 0)),
        ],
        out_shape=[
            jax.ShapeDtypeStruct((B, 1, H), f32),
            jax.ShapeDtypeStruct((B, 1, 128), f32),
        ],
    )(second_capsule, class_capsule, y2, wsel3, fc1_w, fc1_b2)

    BB = 8
    out = pl.pallas_call(
        _dense_kernel,
        grid=(B // BB,),
        in_specs=[
            pl.BlockSpec((BB, M, H), lambda b: (b, 0, 0)),
            pl.BlockSpec((BB, 1, H), lambda b: (b, 0, 0)),
            pl.BlockSpec((BB, 1, 128), lambda b: (b, 0, 0)),
            pl.BlockSpec((BB, M, 1), lambda b: (b, 0, 0)),
        ],
        out_specs=pl.BlockSpec((BB, M, M), lambda b: (b, 0, 0)),
        out_shape=jax.ShapeDtypeStruct((B, M, M), f32),
        compiler_params=pltpu.CompilerParams(
            dimension_semantics=("arbitrary",)),
    )(first_capsule, v_all, s_all, mask3)
    return out


# speedup vs baseline: 1.4712x; 1.4523x over previous
import jax
import jax.numpy as jnp
from jax import lax
from jax.experimental import pallas as pl
from jax.experimental.pallas import tpu as pltpu

N_DIM = 128
HIDDEN = 256
K = 8
EPS = 1e-11
BB = 8


def _preamble(sc2_ref, cls_ref, y_ref, w_ref, b_ref, v_s, s_s):
    f32 = jnp.float32
    B, N2, D = sc2_ref.shape
    NC = cls_ref.shape[1]

    sc2 = sc2_ref[...]
    ones_d = jnp.ones((D, 1), f32)
    sq = lax.dot_general(jnp.reshape(sc2 * sc2, (B * N2, D)), ones_d,
                         (((1,), (0,)), ((), ())),
                         preferred_element_type=f32)
    sq3 = jnp.reshape(sq, (B, N2, 1))
    scale3 = sq3 / (1.0 + sq3)
    sc2n = sc2 / jnp.sqrt(sq3 + EPS)

    iota_n = lax.broadcasted_iota(jnp.int32, (B, N2, 1), 1)
    msk = scale3
    keep = jnp.zeros((B, N2, 1), f32)
    for _ in range(K):
        m = jnp.max(msk, axis=1, keepdims=True)
        idx = jnp.min(jnp.where(msk == m, iota_n, N2), axis=1, keepdims=True)
        sel = iota_n == idx
        keep = jnp.where(sel, 1.0, keep)
        msk = jnp.where(sel, -1.0, msk)
    wsel = scale3 * keep

    iota_c = lax.broadcasted_iota(jnp.int32, (B, NC, 1), 1)
    y3 = jnp.reshape(y_ref[...], (B, 1, 1))
    y1h = (iota_c == y3).astype(f32)
    cc_raw = jnp.sum(y1h * cls_ref[...], axis=1)
    sqc = lax.dot_general(cc_raw * cc_raw, ones_d, (((1,), (0,)), ((), ())),
                          preferred_element_type=f32)
    ccn = cc_raw / jnp.sqrt(sqc + EPS)
    scale1 = sqc / (1.0 + sqc)

    fw = w_ref[...]
    bias = b_ref[...]
    sproc = jnp.maximum(
        lax.dot_general(jnp.reshape(sc2n, (B * N2, D)), fw,
                        (((1,), (1,)), ((), ())),
                        preferred_element_type=f32) + bias, 0.0)
    s1 = jnp.maximum(
        lax.dot_general(ccn, fw, (((1,), (1,)), ((), ())),
                        preferred_element_type=f32) + bias, 0.0)

    ones_h = jnp.ones((HIDDEN, 1), f32)
    rowsq = lax.dot_general(sproc * sproc, ones_h, (((1,), (0,)), ((), ())),
                            preferred_element_type=f32)
    s1sq = lax.dot_general(s1 * s1, ones_h, (((1,), (0,)), ((), ())),
                           preferred_element_type=f32)

    sproc3 = jnp.reshape(sproc, (B, N2, HIDDEN))
    rowsq3 = jnp.reshape(rowsq, (B, N2, 1))

    v = jnp.sum(wsel * sproc3, axis=1) + scale1 * s1
    Wt = jnp.sum(wsel, axis=1) + scale1
    c = jnp.sum(wsel * rowsq3, axis=1) + scale1 * s1sq

    v_s[...] = v
    pad = jnp.zeros((B, 126), f32)
    s_s[...] = jnp.concatenate([Wt, c, pad], axis=1)


def _fused_kernel(sc2_ref, cls_ref, y_ref, w_ref, b_ref, fcap_ref, mask_ref,
                  out_ref, v_s, s_s):
    f32 = jnp.float32
    t = pl.program_id(0)

    @pl.when(t == 0)
    def _():
        _preamble(sc2_ref, cls_ref, y_ref, w_ref, b_ref, v_s, s_s)

    @pl.when(t > 0)
    def _():
        for i in range(BB):
            row = (t - 1) * BB + i
            F = jnp.maximum(fcap_ref[i], 0.0)
            v = v_s[pl.ds(row, 1), :]
            srow = s_s[pl.ds(row, 1), :]
            W = srow[0:1, 0:1]
            c = srow[0:1, 1:2]

            G = lax.dot_general(F, F, (((1,), (1,)), ((), ())),
                                preferred_element_type=f32)
            u_col = lax.dot_general(F, v, (((1,), (1,)), ((), ())),
                                    preferred_element_type=f32)
            u_row = lax.dot_general(v, F, (((1,), (1,)), ((), ())),
                                    preferred_element_type=f32)

            adj = W * G + u_col + u_row + c
            out_ref[i] = adj * mask_ref[i]


def kernel(first_capsule, second_capsule, class_capsule, fc1_w, fc1_b, y,
           first_capsule_mask):
    B, M, H = first_capsule.shape
    N2 = second_capsule.shape[1]
    NC = class_capsule.shape[1]
    f32 = jnp.float32

    y2 = y.astype(jnp.int32).reshape(B, 1)
    mask3 = first_capsule_mask.astype(f32).reshape(B, M, 1)
    fc1_b2 = fc1_b.reshape(1, H)

    def dmap(t):
        td = jnp.maximum(t - 1, 0)
        return (td, 0, 0)

    out = pl.pallas_call(
        _fused_kernel,
        grid=(1 + B // BB,),
        in_specs=[
            pl.BlockSpec(second_capsule.shape, lambda t: (0, 0, 0)),
            pl.BlockSpec(class_capsule.shape, lambda t: (0, 0, 0)),
            pl.BlockSpec((B, 1), lambda t: (0, 0)),
            pl.BlockSpec(fc1_w.shape, lambda t: (0, 0)),
            pl.BlockSpec(fc1_b2.shape, lambda t: (0, 0)),
            pl.BlockSpec((BB, M, H), dmap),
            pl.BlockSpec((BB, M, 1), dmap),
        ],
        out_specs=pl.BlockSpec((BB, M, M), dmap),
        out_shape=jax.ShapeDtypeStruct((B, M, M), f32),
        scratch_shapes=[
            pltpu.VMEM((B, HIDDEN), f32),
            pltpu.VMEM((B, 128), f32),
        ],
        compiler_params=pltpu.CompilerParams(
            dimension_semantics=("arbitrary",)),
    )(second_capsule, class_capsule, y2, fc1_w, fc1_b2, first_capsule, mask3)
    return out
